# initial kernel scaffold (unmeasured)
import jax
import jax.numpy as jnp
from jax import lax
from jax.experimental import pallas as pl
from jax.experimental.pallas import tpu as pltpu

N_DEV = 8


def kernel(x, w_mat, scale_x, scale_w):
    M, _ = x.shape
    _, N = w_mat.shape
    CH = M // N_DEV

    scale = (scale_x * scale_w).reshape(1, 1)

    def body(x_ref, w_ref, scale_ref, out_ref, acc_ref, rs_comm, y_ref,
             rs_send_sems, rs_recv_sems, ag_send_sems, ag_recv_sems,
             local_sem, credit_sem):
        my = lax.axis_index("i")
        left = lax.rem(my + N_DEV - 1, N_DEV)
        right = lax.rem(my + 1, N_DEV)

        barrier_sem = pltpu.get_barrier_semaphore()
        for nbr in (left, right):
            pl.semaphore_signal(barrier_sem, inc=1, device_id=(nbr,),
                                device_id_type=pl.DeviceIdType.MESH)
        pl.semaphore_wait(barrier_sem, 2)

        for c in range(N_DEV):
            acc_ref[pl.ds(c * CH, CH), :] = lax.dot_general(
                x_ref[pl.ds(c * CH, CH), :], w_ref[:, :],
                (((1,), (0,)), ((), ())),
                preferred_element_type=jnp.int32)

        for s in range(N_DEV - 1):
            slot = s % 2
            if s >= 2:
                pl.semaphore_wait(credit_sem, 1)
            send_c = lax.rem(my - s + N_DEV, N_DEV)
            rdma = pltpu.make_async_remote_copy(
                src_ref=acc_ref.at[pl.ds(send_c * CH, CH), :],
                dst_ref=rs_comm.at[slot],
                send_sem=rs_send_sems.at[slot],
                recv_sem=rs_recv_sems.at[slot],
                device_id=(right,),
                device_id_type=pl.DeviceIdType.MESH)
            rdma.start()
            rdma.wait()
            recv_c = lax.rem(my - s - 1 + N_DEV, N_DEV)
            roff = recv_c * CH
            acc_ref[pl.ds(roff, CH), :] = (
                acc_ref[pl.ds(roff, CH), :] + rs_comm[slot])
            if s <= N_DEV - 4:
                pl.semaphore_signal(credit_sem, inc=1, device_id=(left,),
                                    device_id_type=pl.DeviceIdType.MESH)

        own = lax.rem(my + 1, N_DEV)
        ooff = own * CH
        y = acc_ref[pl.ds(ooff, CH), :].astype(jnp.float32) * scale_ref[0, 0]
        yc = jnp.clip(y, -60.0, 60.0)
        y_ref[...] = y / (1.0 + jnp.exp(-yc))
        copy = pltpu.make_async_copy(
            y_ref, out_ref.at[pl.ds(ooff, CH), :], local_sem)
        copy.start()
        copy.wait()

        for t in range(N_DEV - 1):
            send_c = lax.rem(my + 1 - t + N_DEV, N_DEV)
            soff = send_c * CH
            rdma = pltpu.make_async_remote_copy(
                src_ref=out_ref.at[pl.ds(soff, CH), :],
                dst_ref=out_ref.at[pl.ds(soff, CH), :],
                send_sem=ag_send_sems.at[t],
                recv_sem=ag_recv_sems.at[t],
                device_id=(right,),
                device_id_type=pl.DeviceIdType.MESH)
            rdma.start()
            rdma.wait()

    return pl.pallas_call(
        body,
        out_shape=jax.ShapeDtypeStruct((M, N), jnp.float32),
        in_specs=[
            pl.BlockSpec(memory_space=pltpu.MemorySpace.VMEM),
            pl.BlockSpec(memory_space=pltpu.MemorySpace.VMEM),
            pl.BlockSpec(memory_space=pltpu.MemorySpace.SMEM),
        ],
        out_specs=pl.BlockSpec(memory_space=pltpu.MemorySpace.ANY),
        scratch_shapes=[
            pltpu.VMEM((M, N), jnp.int32),
            pltpu.VMEM((2, CH, N), jnp.int32),
            pltpu.VMEM((CH, N), jnp.float32),
            pltpu.SemaphoreType.DMA((2,)),
            pltpu.SemaphoreType.DMA((2,)),
            pltpu.SemaphoreType.DMA((N_DEV - 1,)),
            pltpu.SemaphoreType.DMA((N_DEV - 1,)),
            pltpu.SemaphoreType.DMA,
            pltpu.SemaphoreType.REGULAR,
        ],
        compiler_params=pltpu.CompilerParams(collective_id=0),
    )(x, w_mat, scale)


# baseline (device time: 700816 ns/iter reference)
import jax
import jax.numpy as jnp
from jax import lax
from jax.experimental import pallas as pl
from jax.experimental.pallas import tpu as pltpu

N_DEV = 8


def kernel(x, w_mat, scale_x, scale_w):
    M, _ = x.shape
    _, N = w_mat.shape
    CH = M // N_DEV

    scale = (scale_x * scale_w).reshape(1, 1)

    def body(x_ref, w_ref, scale_ref, out_ref, acc_ref, rs_comm, y_ref,
             rs_send_sems, rs_recv_sems, ag_send_sems, ag_recv_sems,
             local_sem, credit_sem):
        my = lax.axis_index("i")
        left = lax.rem(my + N_DEV - 1, N_DEV)
        right = lax.rem(my + 1, N_DEV)

        barrier_sem = pltpu.get_barrier_semaphore()
        for nbr in (left, right):
            pl.semaphore_signal(barrier_sem, inc=1, device_id=(nbr,),
                                device_id_type=pl.DeviceIdType.MESH)
        pl.semaphore_wait(barrier_sem, 2)

        for c in range(N_DEV):
            acc_ref[pl.ds(c * CH, CH), :] = lax.dot_general(
                x_ref[pl.ds(c * CH, CH), :], w_ref[:, :],
                (((1,), (0,)), ((), ())),
                preferred_element_type=jnp.int32)

        for s in range(N_DEV - 1):
            slot = s % 2
            if s >= 2:
                pl.semaphore_wait(credit_sem, 1)
            send_c = lax.rem(my - s + N_DEV, N_DEV)
            rdma = pltpu.make_async_remote_copy(
                src_ref=acc_ref.at[pl.ds(send_c * CH, CH), :],
                dst_ref=rs_comm.at[slot],
                send_sem=rs_send_sems.at[slot],
                recv_sem=rs_recv_sems.at[slot],
                device_id=(right,),
                device_id_type=pl.DeviceIdType.MESH)
            rdma.start()
            rdma.wait()
            recv_c = lax.rem(my - s - 1 + N_DEV, N_DEV)
            roff = recv_c * CH
            acc_ref[pl.ds(roff, CH), :] = (
                acc_ref[pl.ds(roff, CH), :] + rs_comm[slot])
            if s <= N_DEV - 4:
                pl.semaphore_signal(credit_sem, inc=1, device_id=(left,),
                                    device_id_type=pl.DeviceIdType.MESH)

        own = lax.rem(my + 1, N_DEV)
        ooff = own * CH
        y = acc_ref[pl.ds(ooff, CH), :].astype(jnp.float32) * scale_ref[0, 0]
        yc = jnp.clip(y, -60.0, 60.0)
        y_ref[...] = y / (1.0 + jnp.exp(-yc))
        copy = pltpu.make_async_copy(
            y_ref, out_ref.at[pl.ds(ooff, CH), :], local_sem)
        copy.start()
        copy.wait()

        for t in range(N_DEV - 1):
            send_c = lax.rem(my + 1 - t + N_DEV, N_DEV)
            soff = send_c * CH
            rdma = pltpu.make_async_remote_copy(
                src_ref=out_ref.at[pl.ds(soff, CH), :],
                dst_ref=out_ref.at[pl.ds(soff, CH), :],
                send_sem=ag_send_sems.at[t],
                recv_sem=ag_recv_sems.at[t],
                device_id=(right,),
                device_id_type=pl.DeviceIdType.MESH)
            rdma.start()
            rdma.wait()

    return pl.pallas_call(
        body,
        out_shape=jax.ShapeDtypeStruct((M, N), jnp.float32),
        in_specs=[
            pl.BlockSpec(memory_space=pltpu.MemorySpace.VMEM),
            pl.BlockSpec(memory_space=pltpu.MemorySpace.VMEM),
            pl.BlockSpec(memory_space=pltpu.MemorySpace.SMEM),
        ],
        out_specs=pl.BlockSpec(memory_space=pl.ANY),
        scratch_shapes=[
            pltpu.VMEM((M, N), jnp.int32),
            pltpu.VMEM((2, CH, N), jnp.int32),
            pltpu.VMEM((CH, N), jnp.float32),
            pltpu.SemaphoreType.DMA((2,)),
            pltpu.SemaphoreType.DMA((2,)),
            pltpu.SemaphoreType.DMA((N_DEV - 1,)),
            pltpu.SemaphoreType.DMA((N_DEV - 1,)),
            pltpu.SemaphoreType.DMA,
            pltpu.SemaphoreType.REGULAR,
        ],
        compiler_params=pltpu.CompilerParams(
            collective_id=0, vmem_limit_bytes=60 * 1024 * 1024),
    )(x, w_mat, scale)


# device time: 391844 ns/iter; 1.7885x vs baseline; 1.7885x over previous
import jax
import jax.numpy as jnp
from jax import lax
from jax.experimental import pallas as pl
from jax.experimental.pallas import tpu as pltpu

N_DEV = 8


def kernel(x, w_mat, scale_x, scale_w):
    M, _ = x.shape
    _, N = w_mat.shape
    CH = M // N_DEV
    NH = N // 2

    scale = (scale_x * scale_w).reshape(1, 1)

    def body(x_ref, w_ref, scale_ref, out_ref, acc_ref, comm_r, comm_l,
             y_ref, rs_send_r, rs_recv_r, rs_send_l, rs_recv_l,
             ag_send_r, ag_recv_r, ag_send_l, ag_recv_l,
             local_sems, credit_r, credit_l):
        my = lax.axis_index("i")
        left = lax.rem(my + N_DEV - 1, N_DEV)
        right = lax.rem(my + 1, N_DEV)

        barrier_sem = pltpu.get_barrier_semaphore()
        for nbr in (left, right):
            pl.semaphore_signal(barrier_sem, inc=1, device_id=(nbr,),
                                device_id_type=pl.DeviceIdType.MESH)
        pl.semaphore_wait(barrier_sem, 2)

        for c in range(N_DEV):
            acc_ref[pl.ds(c * CH, CH), :] = lax.dot_general(
                x_ref[pl.ds(c * CH, CH), :], w_ref[:, :],
                (((1,), (0,)), ((), ())),
                preferred_element_type=jnp.int32)

        def rs_rdma(s, slot):
            send_r = lax.rem(my - s + N_DEV, N_DEV)
            send_l = lax.rem(my + s, N_DEV)
            r = pltpu.make_async_remote_copy(
                src_ref=acc_ref.at[pl.ds(send_r * CH, CH), 0:NH],
                dst_ref=comm_r.at[slot],
                send_sem=rs_send_r.at[slot], recv_sem=rs_recv_r.at[slot],
                device_id=(right,), device_id_type=pl.DeviceIdType.MESH)
            l = pltpu.make_async_remote_copy(
                src_ref=acc_ref.at[pl.ds(send_l * CH, CH), NH:N],
                dst_ref=comm_l.at[slot],
                send_sem=rs_send_l.at[slot], recv_sem=rs_recv_l.at[slot],
                device_id=(left,), device_id_type=pl.DeviceIdType.MESH)
            return r, l

        for s in range(N_DEV - 1):
            slot = s % 2
            if s >= 2:
                pl.semaphore_wait(credit_r, 1)
                pl.semaphore_wait(credit_l, 1)
            rdma_r, rdma_l = rs_rdma(s, slot)
            rdma_r.start()
            rdma_l.start()
            rdma_r.wait()
            recv_r = lax.rem(my - s - 1 + N_DEV, N_DEV)
            roff = recv_r * CH
            acc_ref[pl.ds(roff, CH), 0:NH] = (
                acc_ref[pl.ds(roff, CH), 0:NH] + comm_r[slot])
            rdma_l.wait()
            recv_l = lax.rem(my + s + 1, N_DEV)
            loff = recv_l * CH
            acc_ref[pl.ds(loff, CH), NH:N] = (
                acc_ref[pl.ds(loff, CH), NH:N] + comm_l[slot])
            if s <= N_DEV - 4:
                pl.semaphore_signal(credit_r, inc=1, device_id=(left,),
                                    device_id_type=pl.DeviceIdType.MESH)
                pl.semaphore_signal(credit_l, inc=1, device_id=(right,),
                                    device_id_type=pl.DeviceIdType.MESH)

        own_r = lax.rem(my + 1, N_DEV)
        own_l = lax.rem(my + N_DEV - 1, N_DEV)
        s0 = scale_ref[0, 0]
        yr = acc_ref[pl.ds(own_r * CH, CH), 0:NH].astype(jnp.float32) * s0
        y_ref[:, 0:NH] = yr / (1.0 + jnp.exp(-jnp.clip(yr, -60.0, 60.0)))
        yl = acc_ref[pl.ds(own_l * CH, CH), NH:N].astype(jnp.float32) * s0
        y_ref[:, NH:N] = yl / (1.0 + jnp.exp(-jnp.clip(yl, -60.0, 60.0)))
        cp_r = pltpu.make_async_copy(
            y_ref.at[:, 0:NH], out_ref.at[pl.ds(own_r * CH, CH), 0:NH],
            local_sems.at[0])
        cp_l = pltpu.make_async_copy(
            y_ref.at[:, NH:N], out_ref.at[pl.ds(own_l * CH, CH), NH:N],
            local_sems.at[1])
        cp_r.start()
        cp_l.start()
        cp_r.wait()
        cp_l.wait()

        for t in range(N_DEV - 1):
            send_r = lax.rem(my + 1 - t + N_DEV, N_DEV)
            soff_r = send_r * CH
            rdma_r = pltpu.make_async_remote_copy(
                src_ref=out_ref.at[pl.ds(soff_r, CH), 0:NH],
                dst_ref=out_ref.at[pl.ds(soff_r, CH), 0:NH],
                send_sem=ag_send_r.at[t], recv_sem=ag_recv_r.at[t],
                device_id=(right,), device_id_type=pl.DeviceIdType.MESH)
            send_l = lax.rem(my + N_DEV - 1 + t, N_DEV)
            soff_l = send_l * CH
            rdma_l = pltpu.make_async_remote_copy(
                src_ref=out_ref.at[pl.ds(soff_l, CH), NH:N],
                dst_ref=out_ref.at[pl.ds(soff_l, CH), NH:N],
                send_sem=ag_send_l.at[t], recv_sem=ag_recv_l.at[t],
                device_id=(left,), device_id_type=pl.DeviceIdType.MESH)
            rdma_r.start()
            rdma_l.start()
            rdma_r.wait()
            rdma_l.wait()

    return pl.pallas_call(
        body,
        out_shape=jax.ShapeDtypeStruct((M, N), jnp.float32),
        in_specs=[
            pl.BlockSpec(memory_space=pltpu.MemorySpace.VMEM),
            pl.BlockSpec(memory_space=pltpu.MemorySpace.VMEM),
            pl.BlockSpec(memory_space=pltpu.MemorySpace.SMEM),
        ],
        out_specs=pl.BlockSpec(memory_space=pl.ANY),
        scratch_shapes=[
            pltpu.VMEM((M, N), jnp.int32),
            pltpu.VMEM((2, CH, NH), jnp.int32),
            pltpu.VMEM((2, CH, NH), jnp.int32),
            pltpu.VMEM((CH, N), jnp.float32),
            pltpu.SemaphoreType.DMA((2,)),
            pltpu.SemaphoreType.DMA((2,)),
            pltpu.SemaphoreType.DMA((2,)),
            pltpu.SemaphoreType.DMA((2,)),
            pltpu.SemaphoreType.DMA((N_DEV - 1,)),
            pltpu.SemaphoreType.DMA((N_DEV - 1,)),
            pltpu.SemaphoreType.DMA((N_DEV - 1,)),
            pltpu.SemaphoreType.DMA((N_DEV - 1,)),
            pltpu.SemaphoreType.DMA((2,)),
            pltpu.SemaphoreType.REGULAR,
            pltpu.SemaphoreType.REGULAR,
        ],
        compiler_params=pltpu.CompilerParams(
            collective_id=0, vmem_limit_bytes=60 * 1024 * 1024),
    )(x, w_mat, scale)


# device time: 361611 ns/iter; 1.9380x vs baseline; 1.0836x over previous
import jax
import jax.numpy as jnp
from jax import lax
from jax.experimental import pallas as pl
from jax.experimental.pallas import tpu as pltpu

N_DEV = 8
N_CHAIN = 4


def kernel(x, w_mat, scale_x, scale_w):
    M, _ = x.shape
    _, N = w_mat.shape
    CH = M // N_DEV
    ST = N // N_CHAIN

    scale = (scale_x * scale_w).reshape(1, 1)

    CHAIN_DIR = (+1, -1, +1, -1)
    CHAIN_COL = (0, 2 * ST, ST, 3 * ST)

    def body(x_ref, w_ref, scale_ref, out_ref, acc_ref, comm, y_ref,
             rs_send, rs_recv, ag_send, ag_recv, local_sems, credits):
        my = lax.axis_index("i")
        left = lax.rem(my + N_DEV - 1, N_DEV)
        right = lax.rem(my + 1, N_DEV)

        def dst_of(c):
            return right if CHAIN_DIR[c] > 0 else left

        def src_of(c):
            return left if CHAIN_DIR[c] > 0 else right

        def chunk_off(base, delta):
            return lax.rem(base + delta + 2 * N_DEV, N_DEV) * CH

        barrier_sem = pltpu.get_barrier_semaphore()
        for nbr in (left, right):
            pl.semaphore_signal(barrier_sem, inc=1, device_id=(nbr,),
                                device_id_type=pl.DeviceIdType.MESH)
        pl.semaphore_wait(barrier_sem, 2)

        for c in range(N_DEV):
            acc_ref[pl.ds(c * CH, CH), :] = lax.dot_general(
                x_ref[pl.ds(c * CH, CH), :], w_ref[:, :],
                (((1,), (0,)), ((), ())),
                preferred_element_type=jnp.int32)

        def rs_desc(c, s):
            slot = s % 2
            d = CHAIN_DIR[c]
            col = CHAIN_COL[c]
            return pltpu.make_async_remote_copy(
                src_ref=acc_ref.at[pl.ds(chunk_off(my, -d * s), CH),
                                   col:col + ST],
                dst_ref=comm.at[c, slot],
                send_sem=rs_send.at[c, s],
                recv_sem=rs_recv.at[c, slot],
                device_id=(dst_of(c),),
                device_id_type=pl.DeviceIdType.MESH)

        def rs_start(c, s):
            if s >= 2:
                pl.semaphore_wait(credits.at[c], 1)
            rs_desc(c, s).start()

        for c in range(N_CHAIN):
            rs_start(c, 0)
        for s in range(N_DEV - 1):
            for c in range(N_CHAIN):
                d = CHAIN_DIR[c]
                col = CHAIN_COL[c]
                slot = s % 2
                rs_desc(c, s).wait_recv()
                roff = chunk_off(my, -d * (s + 1))
                acc_ref[pl.ds(roff, CH), col:col + ST] = (
                    acc_ref[pl.ds(roff, CH), col:col + ST] + comm[c, slot])
                if s <= N_DEV - 4:
                    pl.semaphore_signal(
                        credits.at[c], inc=1, device_id=(src_of(c),),
                        device_id_type=pl.DeviceIdType.MESH)
                if s < N_DEV - 2:
                    rs_start(c, s + 1)
        for c in range(N_CHAIN):
            for s in range(N_DEV - 1):
                rs_desc(c, s).wait_send()

        s0 = scale_ref[0, 0]
        for c in range(N_CHAIN):
            d = CHAIN_DIR[c]
            col = CHAIN_COL[c]
            ooff = chunk_off(my, d)
            yq = acc_ref[pl.ds(ooff, CH), col:col + ST].astype(
                jnp.float32) * s0
            y_ref[:, col:col + ST] = (
                yq / (1.0 + jnp.exp(-jnp.clip(yq, -60.0, 60.0))))
            pltpu.make_async_copy(
                y_ref.at[:, col:col + ST],
                out_ref.at[pl.ds(ooff, CH), col:col + ST],
                local_sems.at[c]).start()
        for c in range(N_CHAIN):
            d = CHAIN_DIR[c]
            col = CHAIN_COL[c]
            ooff = chunk_off(my, d)
            pltpu.make_async_copy(
                y_ref.at[:, col:col + ST],
                out_ref.at[pl.ds(ooff, CH), col:col + ST],
                local_sems.at[c]).wait()

        def ag_desc(c, t):
            d = CHAIN_DIR[c]
            col = CHAIN_COL[c]
            soff = chunk_off(my, d - d * t)
            return pltpu.make_async_remote_copy(
                src_ref=out_ref.at[pl.ds(soff, CH), col:col + ST],
                dst_ref=out_ref.at[pl.ds(soff, CH), col:col + ST],
                send_sem=ag_send.at[c, t],
                recv_sem=ag_recv.at[c, t],
                device_id=(dst_of(c),),
                device_id_type=pl.DeviceIdType.MESH)

        for c in range(N_CHAIN):
            ag_desc(c, 0).start()
        for t in range(N_DEV - 1):
            for c in range(N_CHAIN):
                ag_desc(c, t).wait_recv()
                if t < N_DEV - 2:
                    ag_desc(c, t + 1).start()
        for c in range(N_CHAIN):
            for t in range(N_DEV - 1):
                ag_desc(c, t).wait_send()

    return pl.pallas_call(
        body,
        out_shape=jax.ShapeDtypeStruct((M, N), jnp.float32),
        in_specs=[
            pl.BlockSpec(memory_space=pltpu.MemorySpace.VMEM),
            pl.BlockSpec(memory_space=pltpu.MemorySpace.VMEM),
            pl.BlockSpec(memory_space=pltpu.MemorySpace.SMEM),
        ],
        out_specs=pl.BlockSpec(memory_space=pl.ANY),
        scratch_shapes=[
            pltpu.VMEM((M, N), jnp.int32),
            pltpu.VMEM((N_CHAIN, 2, CH, ST), jnp.int32),
            pltpu.VMEM((CH, N), jnp.float32),
            pltpu.SemaphoreType.DMA((N_CHAIN, N_DEV - 1)),
            pltpu.SemaphoreType.DMA((N_CHAIN, 2)),
            pltpu.SemaphoreType.DMA((N_CHAIN, N_DEV - 1)),
            pltpu.SemaphoreType.DMA((N_CHAIN, N_DEV - 1)),
            pltpu.SemaphoreType.DMA((N_CHAIN,)),
            pltpu.SemaphoreType.REGULAR((N_CHAIN,)),
        ],
        compiler_params=pltpu.CompilerParams(
            collective_id=0, vmem_limit_bytes=60 * 1024 * 1024),
    )(x, w_mat, scale)


# device time: 204642 ns/iter; 3.4246x vs baseline; 1.7670x over previous
import jax
import jax.numpy as jnp
from jax import lax
from jax.experimental import pallas as pl
from jax.experimental.pallas import tpu as pltpu

N_DEV = 8
N_CHAIN = 4


def kernel(x, w_mat, scale_x, scale_w):
    M, _ = x.shape
    _, N = w_mat.shape
    CH = M // N_DEV
    ST = N // N_CHAIN

    scale = (scale_x * scale_w).reshape(1, 1)

    CHAIN_DIR = (+1, -1, +1, -1)
    CHAIN_COL = (0, 2 * ST, ST, 3 * ST)

    def body(x_ref, w_ref, scale_ref, out_ref, acc_ref, rs_comm, rs_sbuf,
             ag_comm, ag_own, stage, rs_send, rs_recv, ag_send, ag_recv,
             own_sems, stage_sems, credits, ag_credits):
        my = lax.axis_index("i")
        left = lax.rem(my + N_DEV - 1, N_DEV)
        right = lax.rem(my + 1, N_DEV)

        def dst_of(c):
            return right if CHAIN_DIR[c] > 0 else left

        def src_of(c):
            return left if CHAIN_DIR[c] > 0 else right

        def chunk_off(delta):
            return lax.rem(my + delta + 2 * N_DEV, N_DEV) * CH

        barrier_sem = pltpu.get_barrier_semaphore()
        for nbr in (left, right):
            pl.semaphore_signal(barrier_sem, inc=1, device_id=(nbr,),
                                device_id_type=pl.DeviceIdType.MESH)
        pl.semaphore_wait(barrier_sem, 2)

        for c in range(2 * N_DEV):
            acc_ref[pl.ds(c * (CH // 2), CH // 2), :] = lax.dot_general(
                x_ref[pl.ds(c * (CH // 2), CH // 2), :], w_ref[:, :],
                (((1,), (0,)), ((), ())),
                preferred_element_type=jnp.int32)

        def rs_desc(c, s):
            slot = s % 2
            return pltpu.make_async_remote_copy(
                src_ref=rs_sbuf.at[c, slot],
                dst_ref=rs_comm.at[c, slot],
                send_sem=rs_send.at[c, s],
                recv_sem=rs_recv.at[c, slot],
                device_id=(dst_of(c),),
                device_id_type=pl.DeviceIdType.MESH)

        for c in range(N_CHAIN):
            col = CHAIN_COL[c]
            rs_sbuf[c, 0] = acc_ref[
                pl.ds(chunk_off(0), CH), col:col + ST].astype(jnp.bfloat16)
            rs_desc(c, 0).start()
        for s in range(N_DEV - 1):
            slot = s % 2
            for c in range(N_CHAIN):
                d = CHAIN_DIR[c]
                col = CHAIN_COL[c]
                rs_desc(c, s).wait_recv()
                roff = chunk_off(-d * (s + 1))
                acc_ref[pl.ds(roff, CH), col:col + ST] = (
                    acc_ref[pl.ds(roff, CH), col:col + ST]
                    + rs_comm[c, slot].astype(jnp.int32))
                if s <= N_DEV - 4:
                    pl.semaphore_signal(
                        credits.at[c], inc=1, device_id=(src_of(c),),
                        device_id_type=pl.DeviceIdType.MESH)
                if s < N_DEV - 2:
                    if s >= 1:
                        rs_desc(c, s - 1).wait_send()
                    rs_sbuf[c, (s + 1) % 2] = acc_ref[
                        pl.ds(roff, CH), col:col + ST].astype(jnp.bfloat16)
                    if s + 1 >= 2:
                        pl.semaphore_wait(credits.at[c], 1)
                    rs_desc(c, s + 1).start()
        for c in range(N_CHAIN):
            rs_desc(c, N_DEV - 3).wait_send()
            rs_desc(c, N_DEV - 2).wait_send()

        s0 = scale_ref[0, 0]
        for c in range(N_CHAIN):
            d = CHAIN_DIR[c]
            col = CHAIN_COL[c]
            ooff = chunk_off(d)
            yq = acc_ref[pl.ds(ooff, CH), col:col + ST].astype(
                jnp.float32) * s0
            sil = yq / (1.0 + jnp.exp(-jnp.clip(yq, -60.0, 60.0)))
            stage[c] = sil
            ag_own[c] = sil.astype(jnp.bfloat16)
            pltpu.make_async_copy(
                stage.at[c], out_ref.at[pl.ds(ooff, CH), col:col + ST],
                own_sems.at[c]).start()

        def ag_desc(c, t):
            src = ag_own.at[c] if t == 0 else ag_comm.at[c, (t - 1) % 3]
            return pltpu.make_async_remote_copy(
                src_ref=src,
                dst_ref=ag_comm.at[c, t % 3],
                send_sem=ag_send.at[c, t],
                recv_sem=ag_recv.at[c, t],
                device_id=(dst_of(c),),
                device_id_type=pl.DeviceIdType.MESH)

        def stage_copy(c, t):
            d = CHAIN_DIR[c]
            col = CHAIN_COL[c]
            goff = chunk_off(-d * t)
            return pltpu.make_async_copy(
                stage.at[c], out_ref.at[pl.ds(goff, CH), col:col + ST],
                stage_sems.at[c])

        for c in range(N_CHAIN):
            ag_desc(c, 0).start()
        for t in range(N_DEV - 1):
            for c in range(N_CHAIN):
                d = CHAIN_DIR[c]
                col = CHAIN_COL[c]
                ag_desc(c, t).wait_recv()
                if t < N_DEV - 2:
                    if t + 1 >= 3:
                        pl.semaphore_wait(ag_credits.at[c], 1)
                    ag_desc(c, t + 1).start()
                if 1 <= t <= 4:
                    ag_desc(c, t).wait_send()
                    pl.semaphore_signal(
                        ag_credits.at[c], inc=1, device_id=(src_of(c),),
                        device_id_type=pl.DeviceIdType.MESH)
                if t == 0:
                    pltpu.make_async_copy(
                        stage.at[c],
                        out_ref.at[pl.ds(chunk_off(d), CH), col:col + ST],
                        own_sems.at[c]).wait()
                else:
                    stage_copy(c, t - 1).wait()
                stage[c] = ag_comm[c, t % 3].astype(jnp.float32)
                stage_copy(c, t).start()
        for c in range(N_CHAIN):
            stage_copy(c, N_DEV - 2).wait()
            for t in (0, N_DEV - 3, N_DEV - 2):
                ag_desc(c, t).wait_send()

    return pl.pallas_call(
        body,
        out_shape=jax.ShapeDtypeStruct((M, N), jnp.float32),
        in_specs=[
            pl.BlockSpec(memory_space=pltpu.MemorySpace.VMEM),
            pl.BlockSpec(memory_space=pltpu.MemorySpace.VMEM),
            pl.BlockSpec(memory_space=pltpu.MemorySpace.SMEM),
        ],
        out_specs=pl.BlockSpec(memory_space=pl.ANY),
        scratch_shapes=[
            pltpu.VMEM((M, N), jnp.int32),
            pltpu.VMEM((N_CHAIN, 2, CH, ST), jnp.bfloat16),
            pltpu.VMEM((N_CHAIN, 2, CH, ST), jnp.bfloat16),
            pltpu.VMEM((N_CHAIN, 3, CH, ST), jnp.bfloat16),
            pltpu.VMEM((N_CHAIN, CH, ST), jnp.bfloat16),
            pltpu.VMEM((N_CHAIN, CH, ST), jnp.float32),
            pltpu.SemaphoreType.DMA((N_CHAIN, N_DEV - 1)),
            pltpu.SemaphoreType.DMA((N_CHAIN, 2)),
            pltpu.SemaphoreType.DMA((N_CHAIN, N_DEV - 1)),
            pltpu.SemaphoreType.DMA((N_CHAIN, N_DEV - 1)),
            pltpu.SemaphoreType.DMA((N_CHAIN,)),
            pltpu.SemaphoreType.DMA((N_CHAIN,)),
            pltpu.SemaphoreType.REGULAR((N_CHAIN,)),
            pltpu.SemaphoreType.REGULAR((N_CHAIN,)),
        ],
        compiler_params=pltpu.CompilerParams(
            collective_id=0, vmem_limit_bytes=60 * 1024 * 1024),
    )(x, w_mat, scale)


# device time: 192748 ns/iter; 3.6359x vs baseline; 1.0617x over previous
import jax
import jax.numpy as jnp
from jax import lax
from jax.experimental import pallas as pl
from jax.experimental.pallas import tpu as pltpu

N_DEV = 8
N_CHAIN = 4


def kernel(x, w_mat, scale_x, scale_w):
    M, _ = x.shape
    _, N = w_mat.shape
    CH = M // N_DEV
    ST = N // N_CHAIN

    scale = (scale_x * scale_w).reshape(1, 1)

    CHAIN_DIR = (+1, -1, +1, -1)
    CHAIN_COL = (0, 2 * ST, ST, 3 * ST)

    def body(x_ref, w_ref, scale_ref, out_ref, acc_ref, rs_comm, rs_sbuf,
             ag_comm, ag_own, stage, rs_send, rs_recv, ag_send, ag_recv,
             own_sems, stage_sems, credits, ag_credits):
        my = lax.axis_index("i")
        left = lax.rem(my + N_DEV - 1, N_DEV)
        right = lax.rem(my + 1, N_DEV)

        def dst_of(c):
            return right if CHAIN_DIR[c] > 0 else left

        def src_of(c):
            return left if CHAIN_DIR[c] > 0 else right

        def chunk_off(delta):
            return lax.rem(my + delta + 2 * N_DEV, N_DEV) * CH

        barrier_sem = pltpu.get_barrier_semaphore()
        for nbr in (left, right):
            pl.semaphore_signal(barrier_sem, inc=1, device_id=(nbr,),
                                device_id_type=pl.DeviceIdType.MESH)
        pl.semaphore_wait(barrier_sem, 2)

        def gemm_chunk(j):
            base = chunk_off(j)
            for k in range(2):
                off = base + k * (CH // 2)
                acc_ref[pl.ds(off, CH // 2), :] = lax.dot_general(
                    x_ref[pl.ds(off, CH // 2), :], w_ref[:, :],
                    (((1,), (0,)), ((), ())),
                    preferred_element_type=jnp.int32)

        def rs_desc(c, s):
            slot = s % 2
            return pltpu.make_async_remote_copy(
                src_ref=rs_sbuf.at[c, slot],
                dst_ref=rs_comm.at[c, slot],
                send_sem=rs_send.at[c, s],
                recv_sem=rs_recv.at[c, slot],
                device_id=(dst_of(c),),
                device_id_type=pl.DeviceIdType.MESH)

        def ag_desc(c, t):
            src = ag_own.at[c] if t == 0 else ag_comm.at[c, (t - 1) % 3]
            return pltpu.make_async_remote_copy(
                src_ref=src,
                dst_ref=ag_comm.at[c, t % 3],
                send_sem=ag_send.at[c, t],
                recv_sem=ag_recv.at[c, t],
                device_id=(dst_of(c),),
                device_id_type=pl.DeviceIdType.MESH)

        gemm_chunk(0)
        for c in range(N_CHAIN):
            col = CHAIN_COL[c]
            rs_sbuf[c, 0] = acc_ref[
                pl.ds(chunk_off(0), CH), col:col + ST].astype(jnp.bfloat16)
            rs_desc(c, 0).start()
        for j in range(1, N_DEV):
            gemm_chunk(j)

        s0 = scale_ref[0, 0]
        for s in range(N_DEV - 1):
            slot = s % 2
            for c in range(N_CHAIN):
                d = CHAIN_DIR[c]
                col = CHAIN_COL[c]
                rs_desc(c, s).wait_recv()
                roff = chunk_off(-d * (s + 1))
                acc_ref[pl.ds(roff, CH), col:col + ST] = (
                    acc_ref[pl.ds(roff, CH), col:col + ST]
                    + rs_comm[c, slot].astype(jnp.int32))
                if s <= N_DEV - 4:
                    pl.semaphore_signal(
                        credits.at[c], inc=1, device_id=(src_of(c),),
                        device_id_type=pl.DeviceIdType.MESH)
                if s < N_DEV - 2:
                    if s >= 1:
                        rs_desc(c, s - 1).wait_send()
                    rs_sbuf[c, (s + 1) % 2] = acc_ref[
                        pl.ds(roff, CH), col:col + ST].astype(jnp.bfloat16)
                    if s + 1 >= 2:
                        pl.semaphore_wait(credits.at[c], 1)
                    rs_desc(c, s + 1).start()
                else:
                    yq = acc_ref[pl.ds(roff, CH), col:col + ST].astype(
                        jnp.float32) * s0
                    sil = yq / (1.0 + jnp.exp(-jnp.clip(yq, -60.0, 60.0)))
                    stage[c] = sil
                    ag_own[c] = sil.astype(jnp.bfloat16)
                    pltpu.make_async_copy(
                        stage.at[c],
                        out_ref.at[pl.ds(roff, CH), col:col + ST],
                        own_sems.at[c]).start()
                    ag_desc(c, 0).start()
        for c in range(N_CHAIN):
            rs_desc(c, N_DEV - 3).wait_send()
            rs_desc(c, N_DEV - 2).wait_send()

        def stage_copy(c, t):
            d = CHAIN_DIR[c]
            col = CHAIN_COL[c]
            goff = chunk_off(-d * t)
            return pltpu.make_async_copy(
                stage.at[c], out_ref.at[pl.ds(goff, CH), col:col + ST],
                stage_sems.at[c])

        for t in range(N_DEV - 1):
            for c in range(N_CHAIN):
                d = CHAIN_DIR[c]
                col = CHAIN_COL[c]
                ag_desc(c, t).wait_recv()
                if t < N_DEV - 2:
                    if t + 1 >= 3:
                        pl.semaphore_wait(ag_credits.at[c], 1)
                    ag_desc(c, t + 1).start()
                if 1 <= t <= 4:
                    ag_desc(c, t).wait_send()
                    pl.semaphore_signal(
                        ag_credits.at[c], inc=1, device_id=(src_of(c),),
                        device_id_type=pl.DeviceIdType.MESH)
                if t == 0:
                    pltpu.make_async_copy(
                        stage.at[c],
                        out_ref.at[pl.ds(chunk_off(d), CH), col:col + ST],
                        own_sems.at[c]).wait()
                else:
                    stage_copy(c, t - 1).wait()
                stage[c] = ag_comm[c, t % 3].astype(jnp.float32)
                stage_copy(c, t).start()
        for c in range(N_CHAIN):
            stage_copy(c, N_DEV - 2).wait()
            for t in (0, N_DEV - 3, N_DEV - 2):
                ag_desc(c, t).wait_send()

    return pl.pallas_call(
        body,
        out_shape=jax.ShapeDtypeStruct((M, N), jnp.float32),
        in_specs=[
            pl.BlockSpec(memory_space=pltpu.MemorySpace.VMEM),
            pl.BlockSpec(memory_space=pltpu.MemorySpace.VMEM),
            pl.BlockSpec(memory_space=pltpu.MemorySpace.SMEM),
        ],
        out_specs=pl.BlockSpec(memory_space=pl.ANY),
        scratch_shapes=[
            pltpu.VMEM((M, N), jnp.int32),
            pltpu.VMEM((N_CHAIN, 2, CH, ST), jnp.bfloat16),
            pltpu.VMEM((N_CHAIN, 2, CH, ST), jnp.bfloat16),
            pltpu.VMEM((N_CHAIN, 3, CH, ST), jnp.bfloat16),
            pltpu.VMEM((N_CHAIN, CH, ST), jnp.bfloat16),
            pltpu.VMEM((N_CHAIN, CH, ST), jnp.float32),
            pltpu.SemaphoreType.DMA((N_CHAIN, N_DEV - 1)),
            pltpu.SemaphoreType.DMA((N_CHAIN, 2)),
            pltpu.SemaphoreType.DMA((N_CHAIN, N_DEV - 1)),
            pltpu.SemaphoreType.DMA((N_CHAIN, N_DEV - 1)),
            pltpu.SemaphoreType.DMA((N_CHAIN,)),
            pltpu.SemaphoreType.DMA((N_CHAIN,)),
            pltpu.SemaphoreType.REGULAR((N_CHAIN,)),
            pltpu.SemaphoreType.REGULAR((N_CHAIN,)),
        ],
        compiler_params=pltpu.CompilerParams(
            collective_id=0, vmem_limit_bytes=60 * 1024 * 1024),
    )(x, w_mat, scale)
